# baseline (speedup 1.0000x reference)
import functools

import jax
import jax.numpy as jnp
from jax import lax
from jax.experimental import pallas as pl
from jax.experimental.pallas import tpu as pltpu
from jax.experimental.pallas import tpu_sc as plsc

_B = 16384
_L = 20
_D = 16
_NW = 32
_BT = _B // _NW
_C = 128
_NCHUNK = _BT // _C
_NBG = _C // 16
_GPC = _C * _L // 128
_L2_NORM = 1e-4

_mesh = plsc.VectorSubcoreMesh(core_axis_name="c", subcore_axis_name="s")


@functools.partial(
    pl.kernel,
    mesh=_mesh,
    compiler_params=pltpu.CompilerParams(
        needs_layout_passes=False, use_tc_tiling_on_sc=False
    ),
    out_type=(
        jax.ShapeDtypeStruct((_B * _L,), jnp.float32),
        jax.ShapeDtypeStruct((_NW * _D,), jnp.float32),
    ),
    scratch_types=[
        pltpu.VMEM((_BT,), jnp.int32),
        pltpu.VMEM((_BT * _L,), jnp.int32),
        pltpu.VMEM((_BT, _D), jnp.float32),
        pltpu.VMEM((_C * _L, _D), jnp.float32),
        pltpu.VMEM((_C * _L,), jnp.float32),
        pltpu.VMEM((_D,), jnp.float32),
        pltpu.SemaphoreType.DMA,
    ],
)
def _sc_mf(users_f, items_f, uemb, iemb, pred_out, l2_out,
           uidx, iidx, urows, irows, predbuf, l2buf, sem):
    wid = lax.axis_index("s") * 2 + lax.axis_index("c")
    lane = jnp.arange(_D, dtype=jnp.int32)
    dsplat = [jnp.full((_D,), d, jnp.int32) for d in range(_D)]

    pltpu.async_copy(users_f.at[pl.ds(wid * _BT, _BT)], uidx, sem).wait()
    pltpu.async_copy(
        items_f.at[pl.ds(wid * _BT * _L, _BT * _L)], iidx, sem
    ).wait()
    u_handles = [
        pltpu.async_copy(
            uemb.at[uidx.at[pl.ds(g * 128, 128)]],
            urows.at[pl.ds(g * 128, 128)], sem,
        )
        for g in range(_BT // 128)
    ]
    for h in u_handles:
        h.wait()

    def chunk_body(c, accs):
        acc_u, acc_i = accs
        handles = [
            pltpu.async_copy(
                iemb.at[iidx.at[pl.ds(c * _C * _L + j * 128, 128)]],
                irows.at[pl.ds(j * 128, 128)], sem,
            )
            for j in range(_GPC)
        ]
        for h in handles:
            h.wait()

        def bg_body(bg, bg_accs):
            a_u, a_i = bg_accs
            rows_u = c * _C + bg * 16 + lane
            u_vecs = [plsc.load_gather(urows, [rows_u, dsplat[d]])
                      for d in range(_D)]
            for d in range(_D):
                a_u = a_u + u_vecs[d] * u_vecs[d]
            rows0 = (bg * 16 + lane) * _L

            def l_body(l, a_i_in):
                rows_l = rows0 + l
                acc = jnp.zeros((_D,), jnp.float32)
                for d in range(_D):
                    iv = plsc.load_gather(irows, [rows_l, dsplat[d]])
                    acc = acc + u_vecs[d] * iv
                    a_i_in = a_i_in + iv * iv
                plsc.store_scatter(predbuf, [rows_l], acc)
                return a_i_in

            a_i = lax.fori_loop(0, _L, l_body, a_i)
            return a_u, a_i

        acc_u, acc_i = lax.fori_loop(0, _NBG, bg_body, (acc_u, acc_i))
        pltpu.async_copy(
            predbuf,
            pred_out.at[pl.ds((wid * _NCHUNK + c) * _C * _L, _C * _L)],
            sem,
        ).wait()
        return acc_u, acc_i

    zero = jnp.zeros((_D,), jnp.float32)
    acc_u, acc_i = lax.fori_loop(0, _NCHUNK, chunk_body, (zero, zero))
    l2buf[...] = acc_i + jnp.float32(_L) * acc_u
    pltpu.async_copy(l2buf, l2_out.at[pl.ds(wid * _D, _D)], sem).wait()


def kernel(users, items, user_embedding, item_embedding):
    users_f = users.reshape(_B).astype(jnp.int32)
    items_f = items.reshape(_B * _L).astype(jnp.int32)
    pred_flat, l2_parts = _sc_mf(users_f, items_f,
                                 user_embedding, item_embedding)
    l2 = _L2_NORM * jnp.sum(l2_parts)
    return pred_flat.reshape(_B, _L), l2
